# Initial kernel scaffold; baseline (speedup 1.0000x reference)
import jax
import jax.numpy as jnp
from jax.experimental import pallas as pl


def kernel(x, edge_index, W1, b1, W2, b2, W3, b3):
    raise NotImplementedError("write your pallas kernel here")


# speedup vs baseline: 17.5388x; 17.5388x over previous
import functools

import jax
import jax.numpy as jnp
from jax import lax
from jax.experimental import pallas as pl
from jax.experimental.pallas import tpu as pltpu
from jax.experimental.pallas import tpu_sc as plsc

_N = 10000
_D = 128
_NC = 2
_NS = 16
_NW = _NC * _NS
_CHUNK = 128
_CPW = 80
_EPW = _CPW * _CHUNK
_EPAD = _NW * _EPW
_NPAD = _N + 112
_SLAB = _NPAD // _NS
_GW = 128

_mesh = plsc.VectorSubcoreMesh(core_axis_name="c", subcore_axis_name="s")


@functools.partial(
    pl.kernel,
    mesh=_mesh,
    out_type=jax.ShapeDtypeStruct((_NC, _NPAD, _GW), jnp.float32),
    scratch_types=[
        pltpu.VMEM((_CPW, _CHUNK), jnp.int32),
        pltpu.VMEM((_CHUNK, _GW), jnp.float32),
        pltpu.VMEM_SHARED((_NPAD, _GW), jnp.float32),
    ],
)
def _deg_kernel(dst_hbm, ones_hbm, zeros_hbm, out_hbm, dst_v, ones_v, acc):
    cid = lax.axis_index("c")
    sid = lax.axis_index("s")
    wid = cid * _NS + sid
    pltpu.sync_copy(dst_hbm.at[wid], dst_v)
    pltpu.sync_copy(ones_hbm, ones_v)
    pltpu.sync_copy(zeros_hbm, acc.at[pl.ds(sid * _SLAB, _SLAB)])
    plsc.subcore_barrier()

    def step(c, carry):
        pltpu.sync_copy(ones_v, acc.at[dst_v.at[c]], add=True)
        return carry

    lax.fori_loop(0, _CPW, step, 0)
    plsc.subcore_barrier()
    pltpu.sync_copy(acc.at[pl.ds(sid * _SLAB, _SLAB)],
                    out_hbm.at[cid, pl.ds(sid * _SLAB, _SLAB)])


@functools.partial(
    pl.kernel,
    mesh=_mesh,
    out_type=jax.ShapeDtypeStruct((_NC, _NPAD, _D), jnp.float32),
    scratch_types=[
        pltpu.VMEM((_CPW, _CHUNK), jnp.int32),
        pltpu.VMEM((_CPW, _CHUNK), jnp.int32),
        pltpu.VMEM((_CHUNK, _D), jnp.float32),
        pltpu.VMEM_SHARED((_NPAD, _D), jnp.float32),
        pltpu.SemaphoreType.DMA,
    ],
)
def _edge_kernel(src_hbm, dst_hbm, hp_hbm, zeros_hbm, out_hbm,
                 src_v, dst_v, rows_v, acc, sem):
    cid = lax.axis_index("c")
    sid = lax.axis_index("s")
    wid = cid * _NS + sid
    pltpu.sync_copy(src_hbm.at[wid], src_v)
    pltpu.sync_copy(dst_hbm.at[wid], dst_v)
    pltpu.sync_copy(zeros_hbm, acc.at[pl.ds(sid * _SLAB, _SLAB)])
    plsc.subcore_barrier()

    def step(c, carry):
        pltpu.async_copy(hp_hbm.at[src_v.at[c]], rows_v, sem).wait()
        pltpu.sync_copy(rows_v, acc.at[dst_v.at[c]], add=True)
        return carry

    lax.fori_loop(0, _CPW, step, 0)
    plsc.subcore_barrier()
    pltpu.sync_copy(acc.at[pl.ds(sid * _SLAB, _SLAB)],
                    out_hbm.at[cid, pl.ds(sid * _SLAB, _SLAB)])


_BLK = 1000


def _dinv_of(deg_ref):
    deg = deg_ref[0, :, 0] + deg_ref[1, :, 0] + 1.0
    return lax.rsqrt(deg)


def _mm_scale_body(deg_ref, x_ref, w_ref, o_ref):
    dinv = _dinv_of(deg_ref)
    h = jnp.dot(x_ref[...], w_ref[...], preferred_element_type=jnp.float32)
    o_ref[...] = h * dinv[:, None]


def _fused_body(deg_ref, a_ref, hp_ref, b_ref, w_ref, o_ref):
    dinv = _dinv_of(deg_ref)
    s = a_ref[0] + a_ref[1] + hp_ref[...]
    xn = jnp.maximum(dinv[:, None] * s + b_ref[...][None, :], 0.0)
    o_ref[...] = jnp.dot(xn, w_ref[...],
                         preferred_element_type=jnp.float32) * dinv[:, None]


def _final_body(deg_ref, a_ref, hp_ref, b_ref, o_ref):
    dinv = _dinv_of(deg_ref)
    s = a_ref[0] + a_ref[1] + hp_ref[...]
    o_ref[...] = dinv[:, None] * s + b_ref[...][None, :]


_deg_spec = pl.BlockSpec((2, _BLK, _GW), lambda i: (0, i, 0))
_agg_spec = pl.BlockSpec((2, _BLK, _D), lambda i: (0, i, 0))
_row_spec = pl.BlockSpec((_BLK, _D), lambda i: (i, 0))
_b_spec = pl.BlockSpec((_D,), lambda i: (0,))
_w_spec = pl.BlockSpec((_D, _D), lambda i: (0, 0))
_out_sds = jax.ShapeDtypeStruct((_N, _D), jnp.float32)

_mm_scale = pl.pallas_call(
    _mm_scale_body,
    grid=(_N // _BLK,),
    in_specs=[_deg_spec, _row_spec, _w_spec],
    out_specs=_row_spec,
    out_shape=_out_sds,
)

_fused = pl.pallas_call(
    _fused_body,
    grid=(_N // _BLK,),
    in_specs=[_deg_spec, _agg_spec, _row_spec, _b_spec, _w_spec],
    out_specs=_row_spec,
    out_shape=_out_sds,
)

_final = pl.pallas_call(
    _final_body,
    grid=(_N // _BLK,),
    in_specs=[_deg_spec, _agg_spec, _row_spec, _b_spec],
    out_specs=_row_spec,
    out_shape=_out_sds,
)


def kernel(x, edge_index, W1, b1, W2, b2, W3, b3):
    src = edge_index[0].astype(jnp.int32)
    dst = edge_index[1].astype(jnp.int32)
    e = src.shape[0]
    pad = _EPAD - e
    pidx = jnp.arange(pad, dtype=jnp.int32)
    src3 = jnp.concatenate([src, pidx % _N]).reshape(_NW, _CPW, _CHUNK)
    dst3 = jnp.concatenate([dst, _N + (pidx % (_NPAD - _N))]).reshape(
        _NW, _CPW, _CHUNK)
    zeros_d = jnp.zeros((_SLAB, _D), jnp.float32)
    zeros_g = jnp.zeros((_SLAB, _GW), jnp.float32)
    ones_g = jnp.ones((_CHUNK, _GW), jnp.float32)

    deg_parts = _deg_kernel(dst3, ones_g, zeros_g)
    h1 = _mm_scale(deg_parts, x, W1)
    a1 = _edge_kernel(src3, dst3, h1, zeros_d)
    h2 = _fused(deg_parts, a1, h1, b1, W2)
    a2 = _edge_kernel(src3, dst3, h2, zeros_d)
    h3 = _fused(deg_parts, a2, h2, b2, W3)
    a3 = _edge_kernel(src3, dst3, h3, zeros_d)
    return _final(deg_parts, a3, h3, b3)
